# baseline (speedup 1.0000x reference)
import functools

import jax
import jax.numpy as jnp
from jax import lax
from jax.experimental import pallas as pl
from jax.experimental.pallas import tpu as pltpu
from jax.experimental.pallas import tpu_sc as plsc

_VOCAB = 100000
_EMB_DIM = 64
_DEFAULT_IDX = 0
_LANES = 16
_GROUP = 8
_NC = 2
_NS = 16


def _emb_gather(table, idx):
    batch, hist = idx.shape
    nw = _NC * _NS
    rows_per_w = batch // nw
    n_groups = rows_per_w // _GROUP
    mesh = plsc.VectorSubcoreMesh(core_axis_name="c", subcore_axis_name="s")

    @functools.partial(
        pl.kernel,
        out_type=jax.ShapeDtypeStruct((batch, hist, _EMB_DIM), jnp.float32),
        mesh=mesh,
        scratch_types=[
            pltpu.VMEM((rows_per_w, hist), jnp.int32),
            pltpu.VMEM((_GROUP, hist, _EMB_DIM), jnp.float32),
            pltpu.SemaphoreType.DMA,
        ],
        compiler_params=pltpu.CompilerParams(use_tc_tiling_on_sc=False),
    )
    def k(table_hbm, idx_hbm, out_hbm, idx_v, rows_v, sem):
        wid = lax.axis_index("c") * _NS + lax.axis_index("s")
        base = wid * rows_per_w
        pltpu.sync_copy(idx_hbm.at[pl.ds(base, rows_per_w)], idx_v)

        def clamp_row(i, carry):
            for start in (0, 16, 32, hist - _LANES):
                sl = pl.ds(start, _LANES)
                v = idx_v[i, sl]
                ok = (v >= 0) & (v < _VOCAB)
                idx_v[i, sl] = jnp.where(ok, v, _DEFAULT_IDX)
            return carry

        lax.fori_loop(0, rows_per_w, clamp_row, 0)

        def group_body(g, carry):
            g0 = g * _GROUP
            copies = [
                pltpu.async_copy(
                    table_hbm.at[idx_v.at[g0 + j]], rows_v.at[j], sem)
                for j in range(_GROUP)
            ]
            for c in copies:
                c.wait()
            pltpu.sync_copy(rows_v, out_hbm.at[pl.ds(base + g0, _GROUP)])
            return carry

        lax.fori_loop(0, n_groups, group_body, 0)

    return k(table, idx)


def kernel(inputs, embeddings):
    return _emb_gather(embeddings, inputs)


# speedup vs baseline: 1.0475x; 1.0475x over previous
import functools

import jax
import jax.numpy as jnp
from jax import lax
from jax.experimental import pallas as pl
from jax.experimental.pallas import tpu as pltpu
from jax.experimental.pallas import tpu_sc as plsc

_VOCAB = 100000
_EMB_DIM = 64
_DEFAULT_IDX = 0
_LANES = 16
_SEG = 128
_GROUP = 5
_NC = 2
_NS = 16
_BB = 256


def _sc_gather(table2, idx):
    n_total = idx.shape[0]
    nw = _NC * _NS
    per_w = n_total // nw
    n_groups = per_w // (_SEG * _GROUP)
    mesh = plsc.VectorSubcoreMesh(core_axis_name="c", subcore_axis_name="s")

    @functools.partial(
        pl.kernel,
        out_type=jax.ShapeDtypeStruct((n_total, 2 * _EMB_DIM), jnp.float32),
        mesh=mesh,
        scratch_types=[
            pltpu.VMEM((per_w,), jnp.int32),
            pltpu.VMEM((_SEG * _GROUP, 2 * _EMB_DIM), jnp.float32),
            pltpu.SemaphoreType.DMA,
        ],
        compiler_params=pltpu.CompilerParams(use_tc_tiling_on_sc=False),
    )
    def k(table_hbm, idx_hbm, x_hbm, idx_v, stage, sem):
        wid = lax.axis_index("c") * _NS + lax.axis_index("s")
        base = wid * per_w
        pltpu.sync_copy(idx_hbm.at[pl.ds(base, per_w)], idx_v)

        def clamp_body(i, carry):
            sl = pl.ds(i * _LANES, _LANES)
            v = idx_v[sl]
            ok = (v >= 0) & (v < _VOCAB)
            idx_v[sl] = jnp.where(ok, v, _DEFAULT_IDX)
            return carry

        lax.fori_loop(0, per_w // _LANES, clamp_body, 0)

        def group_body(g, carry):
            g0 = g * _SEG * _GROUP
            copies = [
                pltpu.async_copy(
                    table_hbm.at[idx_v.at[pl.ds(g0 + j * _SEG, _SEG)]],
                    stage.at[pl.ds(j * _SEG, _SEG)],
                    sem)
                for j in range(_GROUP)
            ]
            for c in copies:
                c.wait()
            pltpu.sync_copy(
                stage, x_hbm.at[pl.ds(base + g0, _SEG * _GROUP)])
            return carry

        lax.fori_loop(0, n_groups, group_body, 0)

    return k(table2, idx)


def _tc_transpose(x):
    batch = 4096
    hist = 50
    grid = batch // _BB

    def body(x_ref, y_ref):
        xb = x_ref[...][:, :_EMB_DIM]
        y_ref[...] = jnp.transpose(
            xb.reshape(_BB, hist, _EMB_DIM), (1, 2, 0))

    return pl.pallas_call(
        body,
        grid=(grid,),
        in_specs=[pl.BlockSpec((_BB * hist, 2 * _EMB_DIM), lambda i: (i, 0))],
        out_specs=pl.BlockSpec((hist, _EMB_DIM, _BB), lambda i: (0, 0, i)),
        out_shape=jax.ShapeDtypeStruct((hist, _EMB_DIM, batch), jnp.float32),
    )(x)


def kernel(inputs, embeddings):
    b, h = inputs.shape
    table2 = jnp.pad(embeddings, ((0, 0), (0, 2 * _EMB_DIM - embeddings.shape[1])))
    x = _sc_gather(table2, inputs.reshape(-1))
    y = _tc_transpose(x)
    return jnp.transpose(y, (2, 0, 1))
